# Initial kernel scaffold; baseline (speedup 1.0000x reference)
import jax
import jax.numpy as jnp
from jax.experimental import pallas as pl


def kernel(x, meta_x, edge_index, meta_edge_index, W_lin, b_lin, W_meta, b_meta, Wc0, bc0, Wc1, bc1, Wg, bg, Wcls, bcls):
    raise NotImplementedError("write your pallas kernel here")


# speedup vs baseline: 17.9047x; 17.9047x over previous
import functools

import jax
import jax.numpy as jnp
from jax import lax
from jax.experimental import pallas as pl
from jax.experimental.pallas import tpu as pltpu
from jax.experimental.pallas import tpu_sc as plsc

NN = 10000
NE = 320000
D = 128
NCLS = 16
NMETA = 100
NEG = 0.2

NCORE = 2
NSUB = 16
NW = NCORE * NSUB
EW = NE // NW
K = 80
NCHUNK = EW // K
SLAB = 624
TAIL0 = SLAB * NSUB
TAILN = NN - TAIL0

_P = lax.Precision.HIGHEST


def _lrelu(t):
    return jnp.where(t > 0, t, NEG * t)


def _tc1_body(x_ref, wl_ref, bl_ref, mx_ref, wm_ref, bm_ref, wc0_ref,
              xw0_ref, mh_ref):
    h = _lrelu(jnp.dot(x_ref[...], wl_ref[...], precision=_P) + bl_ref[...])
    xw0_ref[...] = jnp.dot(h, wc0_ref[...], precision=_P)
    mh_ref[...] = _lrelu(jnp.dot(mx_ref[...], wm_ref[...], precision=_P)
                         + bm_ref[...])


def _tc2_body(dp_ref, xw0_ref, dinv_ref, y0_ref):
    deg = lax.dot_general(dp_ref[...], jnp.ones((NW, 1), jnp.float32),
                          (((0,), (0,)), ((), ())), precision=_P) + 1.0
    dinv = lax.rsqrt(deg)
    dinv_ref[...] = dinv
    y0_ref[...] = dinv * xw0_ref[...]


def _tc3_body(z_ref, y0_ref, dinv_ref, bc0_ref, wc1_ref, y1_ref):
    dinv = dinv_ref[...]
    h1 = _lrelu(dinv * (z_ref[0] + z_ref[1] + y0_ref[...]) + bc0_ref[...])
    y1_ref[...] = dinv * jnp.dot(h1, wc1_ref[...], precision=_P)


def _tc4a_body(z_ref, y1_ref, dinv_ref, bc1_ref, wg_ref, xgb_ref):
    dinv = dinv_ref[...]
    h2 = _lrelu(dinv * (z_ref[0] + z_ref[1] + y1_ref[...]) + bc1_ref[...])
    xgb_ref[...] = jnp.dot(h2, wg_ref[...], precision=_P)


def _tc4b_body(xgb_ref, mh_ref, wg_ref, bg_ref, md_ref, wcls_ref, bcls_ref,
               ob_ref, om_ref):
    xgb = xgb_ref[...]
    xgm = jnp.dot(mh_ref[...], wg_ref[...], precision=_P)
    lanes = lax.broadcasted_iota(jnp.int32, (NN, D), 1) + NN
    p = jnp.where(md_ref[...] == lanes, 1.0, 0.0)
    s = lax.dot_general(p, xgb, (((0,), (0,)), ((), ())), precision=_P)
    cnt = lax.dot_general(p, jnp.ones((NN, 1), jnp.float32),
                          (((0,), (0,)), ((), ())), precision=_P)
    dinv_m = lax.rsqrt(cnt + 1.0)
    hm_b = _lrelu(xgb + bg_ref[...])
    hm_m = _lrelu(dinv_m * s + (dinv_m * dinv_m) * xgm + bg_ref[...])

    def logsoftmax(t):
        m = jnp.max(t, axis=1, keepdims=True)
        return t - m - jnp.log(jnp.sum(jnp.exp(t - m), axis=1, keepdims=True))

    ob_ref[...] = logsoftmax(jnp.dot(hm_b, wcls_ref[...], precision=_P)
                             + bcls_ref[...])
    om_ref[...] = logsoftmax(jnp.dot(hm_m, wcls_ref[...], precision=_P)
                             + bcls_ref[...])


def _f32(*shape):
    return jax.ShapeDtypeStruct(shape, jnp.float32)


_tc1 = pl.pallas_call(_tc1_body, out_shape=(_f32(NN, D), _f32(D, D)))
_tc2 = pl.pallas_call(_tc2_body, out_shape=(_f32(NN, 1), _f32(NN, D)))
_tc3 = pl.pallas_call(_tc3_body, out_shape=_f32(NN, D))
_tc4a = pl.pallas_call(_tc4a_body, out_shape=_f32(NN, D))
_tc4b = pl.pallas_call(_tc4b_body, out_shape=(_f32(NN, NCLS), _f32(D, NCLS)))


_mesh = plsc.VectorSubcoreMesh(core_axis_name="c", subcore_axis_name="s")


@functools.partial(
    pl.kernel,
    out_type=jax.ShapeDtypeStruct((NCORE, NN, D), jnp.float32),
    mesh=_mesh,
    scratch_types=[
        pltpu.VMEM((NCHUNK, K), jnp.int32),
        pltpu.VMEM((NCHUNK, K), jnp.int32),
        pltpu.VMEM((K, D), jnp.float32),
        pltpu.VMEM_SHARED((NN, D), jnp.float32),
    ],
)
def _sc_agg(y_hbm, src_hbm, dst_hbm, z0_hbm, out_hbm,
            src_v, dst_v, rows_v, z_sh):
    c = lax.axis_index("c")
    s = lax.axis_index("s")
    wid = c * NSUB + s
    row0 = s * SLAB
    pltpu.sync_copy(z0_hbm.at[pl.ds(row0, SLAB)], z_sh.at[pl.ds(row0, SLAB)])

    @pl.when(s == NSUB - 1)
    def _():
        pltpu.sync_copy(z0_hbm.at[pl.ds(TAIL0, TAILN)],
                        z_sh.at[pl.ds(TAIL0, TAILN)])

    pltpu.sync_copy(src_hbm.at[wid], src_v)
    pltpu.sync_copy(dst_hbm.at[wid], dst_v)
    plsc.subcore_barrier()

    @pl.loop(0, NCHUNK)
    def _(i):
        pltpu.sync_copy(y_hbm.at[src_v.at[i]], rows_v)
        pltpu.sync_copy(rows_v, z_sh.at[dst_v.at[i]], add=True)

    plsc.subcore_barrier()
    pltpu.sync_copy(z_sh.at[pl.ds(row0, SLAB)],
                    out_hbm.at[c, pl.ds(row0, SLAB)])

    @pl.when(s == NSUB - 1)
    def _():
        pltpu.sync_copy(z_sh.at[pl.ds(TAIL0, TAILN)],
                        out_hbm.at[c, pl.ds(TAIL0, TAILN)])


@functools.partial(
    pl.kernel,
    out_type=jax.ShapeDtypeStruct((NW, NN), jnp.float32),
    mesh=_mesh,
    compiler_params=pltpu.CompilerParams(needs_layout_passes=False),
    scratch_types=[
        pltpu.VMEM((NCHUNK, K), jnp.int32),
        pltpu.VMEM((NN,), jnp.float32),
    ],
)
def _sc_deg(dst_hbm, out_hbm, dst_v, cnt_v):
    c = lax.axis_index("c")
    s = lax.axis_index("s")
    wid = c * NSUB + s
    pltpu.sync_copy(dst_hbm.at[wid], dst_v)
    zero = jnp.zeros((16,), jnp.float32)

    @pl.loop(0, NN // 16)
    def _(i):
        cnt_v[pl.ds(i * 16, 16)] = zero

    ones = jnp.ones((16,), jnp.float32)

    @pl.loop(0, NCHUNK)
    def _(i):
        @pl.loop(0, K // 16)
        def _(j):
            idx = dst_v[i, pl.ds(j * 16, 16)]
            plsc.addupdate_scatter(cnt_v, [idx], ones)

    pltpu.sync_copy(cnt_v, out_hbm.at[wid])


def kernel(x, meta_x, edge_index, meta_edge_index, W_lin, b_lin, W_meta,
           b_meta, Wc0, bc0, Wc1, bc1, Wg, bg, Wcls, bcls):
    src = edge_index[0].astype(jnp.int32).reshape(NW, NCHUNK, K)
    dst = edge_index[1].astype(jnp.int32).reshape(NW, NCHUNK, K)
    meta_dst = meta_edge_index[1].astype(jnp.int32).reshape(NN, 1)
    mx_pad = jnp.pad(meta_x, ((0, D - NMETA), (0, 0)))
    zeros_nd = jnp.zeros((NN, D), jnp.float32)

    deg_p = _sc_deg(dst)
    xw0, mh = _tc1(x, W_lin, b_lin.reshape(1, D), mx_pad, W_meta,
                   b_meta.reshape(1, D), Wc0)
    dinv, y0 = _tc2(deg_p, xw0)
    z0 = _sc_agg(y0, src, dst, zeros_nd)
    y1 = _tc3(z0, y0, dinv, bc0.reshape(1, D), Wc1)
    z1 = _sc_agg(y1, src, dst, zeros_nd)
    xgb = _tc4a(z1, y1, dinv, bc1.reshape(1, D), Wg)
    ob, om = _tc4b(xgb, mh, Wg, bg.reshape(1, D), meta_dst, Wcls,
                   bcls.reshape(1, NCLS))
    return jnp.concatenate([ob, om[:NMETA]], axis=0)


# speedup vs baseline: 25.2534x; 1.4104x over previous
import functools

import jax
import jax.numpy as jnp
from jax import lax
from jax.experimental import pallas as pl
from jax.experimental.pallas import tpu as pltpu
from jax.experimental.pallas import tpu_sc as plsc

NN = 10000
NE = 320000
D = 128
NCLS = 16
NMETA = 100
NEG = 0.2

NCORE = 2
NSUB = 16
NW = NCORE * NSUB
EW = NE // NW
K = 80
NCHUNK = EW // K
SB = 25
NSB = NCHUNK // SB
SLAB = 624
TAIL0 = SLAB * NSUB
TAILN = NN - TAIL0

_P = lax.Precision.HIGHEST


def _lrelu(t):
    return jnp.where(t > 0, t, NEG * t)


def _tc1_body(x_ref, wl_ref, bl_ref, mx_ref, wm_ref, bm_ref, wc0_ref,
              xw0_ref, mh_ref):
    h = _lrelu(jnp.dot(x_ref[...], wl_ref[...], precision=_P) + bl_ref[...])
    xw0_ref[...] = jnp.dot(h, wc0_ref[...], precision=_P)
    mh_ref[...] = _lrelu(jnp.dot(mx_ref[...], wm_ref[...], precision=_P)
                         + bm_ref[...])


def _tc2_body(dp_ref, xw0_ref, dinv_ref, y0_ref):
    deg = lax.dot_general(dp_ref[...], jnp.ones((NW, 1), jnp.float32),
                          (((0,), (0,)), ((), ())), precision=_P) + 1.0
    dinv = lax.rsqrt(deg)
    dinv_ref[...] = dinv
    y0_ref[...] = dinv * xw0_ref[...]


def _tc3_body(z_ref, y0_ref, dinv_ref, bc0_ref, wc1_ref, y1_ref):
    dinv = dinv_ref[...]
    h1 = _lrelu(dinv * (z_ref[0] + z_ref[1] + y0_ref[...]) + bc0_ref[...])
    y1_ref[...] = dinv * jnp.dot(h1, wc1_ref[...], precision=_P)


def _tc4a_body(z_ref, y1_ref, dinv_ref, bc1_ref, wg_ref, xgb_ref):
    dinv = dinv_ref[...]
    h2 = _lrelu(dinv * (z_ref[0] + z_ref[1] + y1_ref[...]) + bc1_ref[...])
    xgb_ref[...] = jnp.dot(h2, wg_ref[...], precision=_P)


def _tc4b_body(xgb_ref, mh_ref, wg_ref, bg_ref, md_ref, wcls_ref, bcls_ref,
               ob_ref, om_ref):
    xgb = xgb_ref[...]
    xgm = jnp.dot(mh_ref[...], wg_ref[...], precision=_P)
    lanes = lax.broadcasted_iota(jnp.int32, (NN, D), 1) + NN
    p = jnp.where(md_ref[...] == lanes, 1.0, 0.0)
    s = lax.dot_general(p, xgb, (((0,), (0,)), ((), ())), precision=_P)
    cnt = lax.dot_general(p, jnp.ones((NN, 1), jnp.float32),
                          (((0,), (0,)), ((), ())), precision=_P)
    dinv_m = lax.rsqrt(cnt + 1.0)
    hm_b = _lrelu(xgb + bg_ref[...])
    hm_m = _lrelu(dinv_m * s + (dinv_m * dinv_m) * xgm + bg_ref[...])

    def logsoftmax(t):
        m = jnp.max(t, axis=1, keepdims=True)
        return t - m - jnp.log(jnp.sum(jnp.exp(t - m), axis=1, keepdims=True))

    ob_ref[...] = logsoftmax(jnp.dot(hm_b, wcls_ref[...], precision=_P)
                             + bcls_ref[...])
    om_ref[...] = logsoftmax(jnp.dot(hm_m, wcls_ref[...], precision=_P)
                             + bcls_ref[...])


def _f32(*shape):
    return jax.ShapeDtypeStruct(shape, jnp.float32)


_tc1 = pl.pallas_call(_tc1_body, out_shape=(_f32(NN, D), _f32(D, D)))
_tc2 = pl.pallas_call(_tc2_body, out_shape=(_f32(NN, 1), _f32(NN, D)))
_tc3 = pl.pallas_call(_tc3_body, out_shape=_f32(NN, D))
_tc4a = pl.pallas_call(_tc4a_body, out_shape=_f32(NN, D))
_tc4b = pl.pallas_call(_tc4b_body, out_shape=(_f32(NN, NCLS), _f32(D, NCLS)))


_mesh = plsc.VectorSubcoreMesh(core_axis_name="c", subcore_axis_name="s")


@functools.partial(
    pl.kernel,
    out_type=jax.ShapeDtypeStruct((NCORE, NN, D), jnp.float32),
    mesh=_mesh,
    scratch_types=[
        pltpu.VMEM((SB, K), jnp.int32),
        pltpu.VMEM((SB, K), jnp.int32),
        pltpu.VMEM((K, D), jnp.float32),
        pltpu.VMEM((K, D), jnp.float32),
        pltpu.VMEM_SHARED((NN, D), jnp.float32),
        pltpu.SemaphoreType.DMA,
        pltpu.SemaphoreType.DMA,
    ],
)
def _sc_agg(y_hbm, src_hbm, dst_hbm, z0_hbm, out_hbm,
            src_v, dst_v, rows0_v, rows1_v, z_sh, sem0, sem1):
    c = lax.axis_index("c")
    s = lax.axis_index("s")
    wid = c * NSUB + s
    row0 = s * SLAB
    pltpu.sync_copy(z0_hbm.at[pl.ds(row0, SLAB)], z_sh.at[pl.ds(row0, SLAB)])

    @pl.when(s == NSUB - 1)
    def _():
        pltpu.sync_copy(z0_hbm.at[pl.ds(TAIL0, TAILN)],
                        z_sh.at[pl.ds(TAIL0, TAILN)])

    plsc.subcore_barrier()

    @pl.loop(0, NSB)
    def _(p):
        pltpu.sync_copy(src_hbm.at[wid, p], src_v)
        pltpu.sync_copy(dst_hbm.at[wid, p], dst_v)
        pltpu.async_copy(y_hbm.at[src_v.at[0]], rows0_v, sem0)

        @pl.loop(0, SB - 1, step=2)
        def _(i):
            pltpu.async_copy(y_hbm.at[src_v.at[i + 1]], rows1_v, sem1)
            pltpu.make_async_copy(y_hbm.at[src_v.at[i]], rows0_v, sem0).wait()
            pltpu.sync_copy(rows0_v, z_sh.at[dst_v.at[i]], add=True)
            pltpu.async_copy(y_hbm.at[src_v.at[i + 2]], rows0_v, sem0)
            pltpu.make_async_copy(y_hbm.at[src_v.at[i + 1]], rows1_v,
                                  sem1).wait()
            pltpu.sync_copy(rows1_v, z_sh.at[dst_v.at[i + 1]], add=True)

        pltpu.make_async_copy(y_hbm.at[src_v.at[SB - 1]], rows0_v,
                              sem0).wait()
        pltpu.sync_copy(rows0_v, z_sh.at[dst_v.at[SB - 1]], add=True)

    plsc.subcore_barrier()
    pltpu.sync_copy(z_sh.at[pl.ds(row0, SLAB)],
                    out_hbm.at[c, pl.ds(row0, SLAB)])

    @pl.when(s == NSUB - 1)
    def _():
        pltpu.sync_copy(z_sh.at[pl.ds(TAIL0, TAILN)],
                        out_hbm.at[c, pl.ds(TAIL0, TAILN)])


@functools.partial(
    pl.kernel,
    out_type=jax.ShapeDtypeStruct((NW, NN), jnp.float32),
    mesh=_mesh,
    compiler_params=pltpu.CompilerParams(needs_layout_passes=False),
    scratch_types=[
        pltpu.VMEM((EW,), jnp.int32),
        pltpu.VMEM((NN,), jnp.float32),
    ],
)
def _sc_deg(dst_hbm, out_hbm, dst_v, cnt_v):
    c = lax.axis_index("c")
    s = lax.axis_index("s")
    wid = c * NSUB + s
    pltpu.sync_copy(dst_hbm.at[wid], dst_v)
    zero = jnp.zeros((16,), jnp.float32)

    @pl.loop(0, NN // 16)
    def _(i):
        cnt_v[pl.ds(i * 16, 16)] = zero

    ones = jnp.ones((16,), jnp.float32)

    @pl.loop(0, EW // 16)
    def _(i):
        idx = dst_v[pl.ds(i * 16, 16)]
        plsc.addupdate_scatter(cnt_v, [idx], ones)

    pltpu.sync_copy(cnt_v, out_hbm.at[wid])


def kernel(x, meta_x, edge_index, meta_edge_index, W_lin, b_lin, W_meta,
           b_meta, Wc0, bc0, Wc1, bc1, Wg, bg, Wcls, bcls):
    src = edge_index[0].astype(jnp.int32).reshape(NW, NSB, SB, K)
    dst = edge_index[1].astype(jnp.int32).reshape(NW, NSB, SB, K)
    meta_dst = meta_edge_index[1].astype(jnp.int32).reshape(NN, 1)
    mx_pad = jnp.pad(meta_x, ((0, D - NMETA), (0, 0)))
    zeros_nd = jnp.zeros((NN, D), jnp.float32)

    deg_p = _sc_deg(edge_index[1].astype(jnp.int32).reshape(NW, EW))
    xw0, mh = _tc1(x, W_lin, b_lin.reshape(1, D), mx_pad, W_meta,
                   b_meta.reshape(1, D), Wc0)
    dinv, y0 = _tc2(deg_p, xw0)
    z0 = _sc_agg(y0, src, dst, zeros_nd)
    y1 = _tc3(z0, y0, dinv, bc0.reshape(1, D), Wc1)
    z1 = _sc_agg(y1, src, dst, zeros_nd)
    xgb = _tc4a(z1, y1, dinv, bc1.reshape(1, D), Wg)
    ob, om = _tc4b(xgb, mh, Wg, bg.reshape(1, D), meta_dst, Wcls,
                   bcls.reshape(1, NCLS))
    return jnp.concatenate([ob, om[:NMETA]], axis=0)


# speedup vs baseline: 27.7038x; 1.0970x over previous
import functools

import jax
import jax.numpy as jnp
from jax import lax
from jax.experimental import pallas as pl
from jax.experimental.pallas import tpu as pltpu
from jax.experimental.pallas import tpu_sc as plsc

NN = 10000
NE = 320000
D = 128
NCLS = 16
NMETA = 100
NEG = 0.2

NCORE = 2
NSUB = 16
NW = NCORE * NSUB
EW = NE // NW
K = 80
NCHUNK = EW // K
SB = 25
NSB = NCHUNK // SB
SLAB = 624
TAIL0 = SLAB * NSUB
TAILN = NN - TAIL0

_P = lax.Precision.HIGHEST


def _lrelu(t):
    return jnp.where(t > 0, t, NEG * t)


def _tc1_body(x_ref, wl_ref, bl_ref, mx_ref, wm_ref, bm_ref, wc0_ref,
              xw0_ref, mh_ref):
    h = _lrelu(jnp.dot(x_ref[...], wl_ref[...], precision=_P) + bl_ref[...])
    xw0_ref[...] = jnp.dot(h, wc0_ref[...], precision=_P)
    mh_ref[...] = _lrelu(jnp.dot(mx_ref[...], wm_ref[...], precision=_P)
                         + bm_ref[...])


def _tc2_body(dp_ref, xw0_ref, dinv_ref, y0_ref):
    deg = lax.dot_general(dp_ref[...], jnp.ones((NW, 1), jnp.float32),
                          (((0,), (0,)), ((), ())), precision=_P) + 1.0
    dinv = lax.rsqrt(deg)
    dinv_ref[...] = dinv
    y0_ref[...] = dinv * xw0_ref[...]


def _tc3_body(z_ref, y0_ref, dinv_ref, bc0_ref, wc1_ref, y1_ref):
    dinv = dinv_ref[...]
    h1 = _lrelu(dinv * (z_ref[0] + z_ref[1] + y0_ref[...]) + bc0_ref[...])
    y1_ref[...] = dinv * jnp.dot(h1, wc1_ref[...], precision=_P)


def _tc4a_body(z_ref, y1_ref, dinv_ref, bc1_ref, wg_ref, xgb_ref):
    dinv = dinv_ref[...]
    h2 = _lrelu(dinv * (z_ref[0] + z_ref[1] + y1_ref[...]) + bc1_ref[...])
    xgb_ref[...] = jnp.dot(h2, wg_ref[...], precision=_P)


def _tc4b_body(xgb_ref, mh_ref, wg_ref, bg_ref, md_ref, wcls_ref, bcls_ref,
               ob_ref, om_ref):
    xgb = xgb_ref[...]
    xgm = jnp.dot(mh_ref[...], wg_ref[...], precision=_P)
    lanes = lax.broadcasted_iota(jnp.int32, (NN, D), 1) + NN
    p = jnp.where(md_ref[...] == lanes, 1.0, 0.0)
    s = lax.dot_general(p, xgb, (((0,), (0,)), ((), ())), precision=_P)
    cnt = lax.dot_general(p, jnp.ones((NN, 1), jnp.float32),
                          (((0,), (0,)), ((), ())), precision=_P)
    dinv_m = lax.rsqrt(cnt + 1.0)
    hm_b = _lrelu(xgb + bg_ref[...])
    hm_m = _lrelu(dinv_m * s + (dinv_m * dinv_m) * xgm + bg_ref[...])

    def logsoftmax(t):
        m = jnp.max(t, axis=1, keepdims=True)
        return t - m - jnp.log(jnp.sum(jnp.exp(t - m), axis=1, keepdims=True))

    ob_ref[...] = logsoftmax(jnp.dot(hm_b, wcls_ref[...], precision=_P)
                             + bcls_ref[...])
    om_ref[...] = logsoftmax(jnp.dot(hm_m, wcls_ref[...], precision=_P)
                             + bcls_ref[...])


def _f32(*shape):
    return jax.ShapeDtypeStruct(shape, jnp.float32)


_tc1 = pl.pallas_call(_tc1_body, out_shape=(_f32(NN, D), _f32(D, D)))
_tc2 = pl.pallas_call(_tc2_body, out_shape=(_f32(NN, 1), _f32(NN, D)))
_tc3 = pl.pallas_call(_tc3_body, out_shape=_f32(NN, D))
_tc4a = pl.pallas_call(_tc4a_body, out_shape=_f32(NN, D))
_tc4b = pl.pallas_call(_tc4b_body, out_shape=(_f32(NN, NCLS), _f32(D, NCLS)))


_mesh = plsc.VectorSubcoreMesh(core_axis_name="c", subcore_axis_name="s")


@functools.partial(
    pl.kernel,
    out_type=jax.ShapeDtypeStruct((NCORE, NN, D), jnp.float32),
    mesh=_mesh,
    scratch_types=[
        pltpu.VMEM((SB, K), jnp.int32),
        pltpu.VMEM((SB, K), jnp.int32),
        pltpu.VMEM((K, D), jnp.float32),
        pltpu.VMEM((K, D), jnp.float32),
        pltpu.VMEM((K, D), jnp.float32),
        pltpu.VMEM_SHARED((NN, D), jnp.float32),
        pltpu.SemaphoreType.DMA,
        pltpu.SemaphoreType.DMA,
        pltpu.SemaphoreType.DMA,
        pltpu.SemaphoreType.DMA,
        pltpu.SemaphoreType.DMA,
        pltpu.SemaphoreType.DMA,
    ],
)
def _sc_agg(y_hbm, src_hbm, dst_hbm, z0_hbm, out_hbm,
            src_v, dst_v, r0, r1, r2, z_sh,
            gs0, gs1, gs2, ss0, ss1, ss2):
    c = lax.axis_index("c")
    s = lax.axis_index("s")
    wid = c * NSUB + s
    row0 = s * SLAB
    pltpu.sync_copy(z0_hbm.at[pl.ds(row0, SLAB)], z_sh.at[pl.ds(row0, SLAB)])

    @pl.when(s == NSUB - 1)
    def _():
        pltpu.sync_copy(z0_hbm.at[pl.ds(TAIL0, TAILN)],
                        z_sh.at[pl.ds(TAIL0, TAILN)])

    plsc.subcore_barrier()

    rows = (r0, r1, r2)
    gsem = (gs0, gs1, gs2)
    ssem = (ss0, ss1, ss2)

    def fire_g(c, b):
        pltpu.async_copy(y_hbm.at[src_v.at[c]], rows[b], gsem[b])

    def wait_g(c, b):
        pltpu.make_async_copy(y_hbm.at[src_v.at[c]], rows[b], gsem[b]).wait()

    def fire_s(c, b):
        pltpu.async_copy(rows[b], z_sh.at[dst_v.at[c]], ssem[b], add=True)

    def wait_s(c, b):
        pltpu.make_async_copy(rows[b], z_sh.at[dst_v.at[c]], ssem[b]).wait()

    @pl.loop(0, NSB)
    def _(p):
        pltpu.sync_copy(src_hbm.at[wid, p], src_v)
        pltpu.sync_copy(dst_hbm.at[wid, p], dst_v)
        fire_g(0, 0)
        fire_g(1, 1)
        wait_g(0, 0)
        fire_s(0, 0)
        fire_g(2, 2)

        @pl.loop(1, SB - 3, step=3)
        def _(t):
            for o, b in ((0, 1), (1, 2), (2, 0)):
                c = t + o
                wait_g(c, b)
                fire_s(c, b)
                wait_s(c - 1, (b - 1) % 3)
                fire_g(c + 2, (b + 2) % 3)

        wait_g(SB - 3, 1)
        fire_s(SB - 3, 1)
        wait_s(SB - 4, 0)
        fire_g(SB - 1, 0)
        wait_g(SB - 2, 2)
        fire_s(SB - 2, 2)
        wait_s(SB - 3, 1)
        wait_g(SB - 1, 0)
        fire_s(SB - 1, 0)
        wait_s(SB - 2, 2)
        wait_s(SB - 1, 0)

    plsc.subcore_barrier()
    pltpu.sync_copy(z_sh.at[pl.ds(row0, SLAB)],
                    out_hbm.at[c, pl.ds(row0, SLAB)])

    @pl.when(s == NSUB - 1)
    def _():
        pltpu.sync_copy(z_sh.at[pl.ds(TAIL0, TAILN)],
                        out_hbm.at[c, pl.ds(TAIL0, TAILN)])


@functools.partial(
    pl.kernel,
    out_type=jax.ShapeDtypeStruct((NW, NN), jnp.float32),
    mesh=_mesh,
    compiler_params=pltpu.CompilerParams(needs_layout_passes=False),
    scratch_types=[
        pltpu.VMEM((EW,), jnp.int32),
        pltpu.VMEM((NN,), jnp.float32),
    ],
)
def _sc_deg(dst_hbm, out_hbm, dst_v, cnt_v):
    c = lax.axis_index("c")
    s = lax.axis_index("s")
    wid = c * NSUB + s
    pltpu.sync_copy(dst_hbm.at[wid], dst_v)
    zero = jnp.zeros((16,), jnp.float32)

    @pl.loop(0, NN // 16)
    def _(i):
        cnt_v[pl.ds(i * 16, 16)] = zero

    ones = jnp.ones((16,), jnp.float32)

    @pl.loop(0, EW // 16)
    def _(i):
        idx = dst_v[pl.ds(i * 16, 16)]
        plsc.addupdate_scatter(cnt_v, [idx], ones)

    pltpu.sync_copy(cnt_v, out_hbm.at[wid])


def kernel(x, meta_x, edge_index, meta_edge_index, W_lin, b_lin, W_meta,
           b_meta, Wc0, bc0, Wc1, bc1, Wg, bg, Wcls, bcls):
    src = edge_index[0].astype(jnp.int32).reshape(NW, NSB, SB, K)
    dst = edge_index[1].astype(jnp.int32).reshape(NW, NSB, SB, K)
    meta_dst = meta_edge_index[1].astype(jnp.int32).reshape(NN, 1)
    mx_pad = jnp.pad(meta_x, ((0, D - NMETA), (0, 0)))
    zeros_nd = jnp.zeros((NN, D), jnp.float32)

    deg_p = _sc_deg(edge_index[1].astype(jnp.int32).reshape(NW, EW))
    xw0, mh = _tc1(x, W_lin, b_lin.reshape(1, D), mx_pad, W_meta,
                   b_meta.reshape(1, D), Wc0)
    dinv, y0 = _tc2(deg_p, xw0)
    z0 = _sc_agg(y0, src, dst, zeros_nd)
    y1 = _tc3(z0, y0, dinv, bc0.reshape(1, D), Wc1)
    z1 = _sc_agg(y1, src, dst, zeros_nd)
    xgb = _tc4a(z1, y1, dinv, bc1.reshape(1, D), Wg)
    ob, om = _tc4b(xgb, mh, Wg, bg.reshape(1, D), meta_dst, Wcls,
                   bcls.reshape(1, NCLS))
    return jnp.concatenate([ob, om[:NMETA]], axis=0)


# speedup vs baseline: 31.5857x; 1.1401x over previous
import functools

import jax
import jax.numpy as jnp
from jax import lax
from jax.experimental import pallas as pl
from jax.experimental.pallas import tpu as pltpu
from jax.experimental.pallas import tpu_sc as plsc

NN = 10000
NE = 320000
D = 128
NCLS = 16
NMETA = 100
NEG = 0.2

NCORE = 2
NSUB = 16
NW = NCORE * NSUB
EW = NE // NW
K = 80
NCHUNK = EW // K
SB = 25
NSB = NCHUNK // SB
SLAB = 624
TAIL0 = SLAB * NSUB
TAILN = NN - TAIL0

_P = lax.Precision.DEFAULT


def _lrelu(t):
    return jnp.where(t > 0, t, NEG * t)


def _tc1_body(x_ref, wl_ref, bl_ref, mx_ref, wm_ref, bm_ref, wc0_ref,
              xw0_ref, mh_ref):
    h = _lrelu(jnp.dot(x_ref[...], wl_ref[...], precision=_P) + bl_ref[...])
    xw0_ref[...] = jnp.dot(h, wc0_ref[...], precision=_P)
    mh_ref[...] = _lrelu(jnp.dot(mx_ref[...], wm_ref[...], precision=_P)
                         + bm_ref[...])


def _tc2_body(dp_ref, xw0_ref, dinv_ref, y0_ref):
    deg = lax.dot_general(dp_ref[...], jnp.ones((NW, 1), jnp.float32),
                          (((0,), (0,)), ((), ())), precision=_P) + 1.0
    dinv = lax.rsqrt(deg)
    dinv_ref[...] = dinv
    y0_ref[...] = dinv * xw0_ref[...]


def _tc3_body(z_ref, y0_ref, dinv_ref, bc0_ref, wc1_ref, y1_ref):
    dinv = dinv_ref[...]
    h1 = _lrelu(dinv * (z_ref[0] + z_ref[1] + y0_ref[...]) + bc0_ref[...])
    y1_ref[...] = dinv * jnp.dot(h1, wc1_ref[...], precision=_P)


def _tc4a_body(z_ref, y1_ref, dinv_ref, bc1_ref, wg_ref, xgb_ref):
    dinv = dinv_ref[...]
    h2 = _lrelu(dinv * (z_ref[0] + z_ref[1] + y1_ref[...]) + bc1_ref[...])
    xgb_ref[...] = jnp.dot(h2, wg_ref[...], precision=_P)


def _tc4b_body(xgb_ref, mh_ref, wg_ref, bg_ref, md_ref, wcls_ref, bcls_ref,
               out_ref):
    xgb = xgb_ref[...]
    xgm = jnp.dot(mh_ref[...], wg_ref[...], precision=_P)
    lanes = lax.broadcasted_iota(jnp.int32, (NN, D), 1) + NN
    p = jnp.where(md_ref[...] == lanes, 1.0, 0.0)
    s = lax.dot_general(p, xgb, (((0,), (0,)), ((), ())), precision=_P)
    cnt = lax.dot_general(p, jnp.ones((NN, 1), jnp.float32),
                          (((0,), (0,)), ((), ())), precision=_P)
    dinv_m = lax.rsqrt(cnt + 1.0)
    hm_b = _lrelu(xgb + bg_ref[...])
    hm_m = _lrelu(dinv_m * s + (dinv_m * dinv_m) * xgm + bg_ref[...])

    def logsoftmax(t):
        m = jnp.max(t, axis=1, keepdims=True)
        return t - m - jnp.log(jnp.sum(jnp.exp(t - m), axis=1, keepdims=True))

    out_ref[0:NN, :] = logsoftmax(jnp.dot(hm_b, wcls_ref[...], precision=_P)
                                  + bcls_ref[...])
    om = logsoftmax(jnp.dot(hm_m, wcls_ref[...], precision=_P)
                    + bcls_ref[...])
    out_ref[NN:NN + NMETA, :] = om[:NMETA]


def _f32(*shape):
    return jax.ShapeDtypeStruct(shape, jnp.float32)


_tc1 = pl.pallas_call(_tc1_body, out_shape=(_f32(NN, D), _f32(D, D)))
_tc2 = pl.pallas_call(_tc2_body, out_shape=(_f32(NN, 1), _f32(NN, D)))
_tc3 = pl.pallas_call(_tc3_body, out_shape=_f32(NN, D))
_tc4a = pl.pallas_call(_tc4a_body, out_shape=_f32(NN, D))
_tc4b = pl.pallas_call(_tc4b_body, out_shape=_f32(NN + NMETA, NCLS))


_mesh = plsc.VectorSubcoreMesh(core_axis_name="c", subcore_axis_name="s")


@functools.partial(
    pl.kernel,
    out_type=jax.ShapeDtypeStruct((NCORE, NN, D), jnp.float32),
    mesh=_mesh,
    scratch_types=[
        pltpu.VMEM((SB, K), jnp.int32),
        pltpu.VMEM((SB, K), jnp.int32),
        pltpu.VMEM((K, D), jnp.float32),
        pltpu.VMEM((K, D), jnp.float32),
        pltpu.VMEM((K, D), jnp.float32),
        pltpu.VMEM_SHARED((NN, D), jnp.float32),
        pltpu.SemaphoreType.DMA,
        pltpu.SemaphoreType.DMA,
        pltpu.SemaphoreType.DMA,
        pltpu.SemaphoreType.DMA,
        pltpu.SemaphoreType.DMA,
        pltpu.SemaphoreType.DMA,
    ],
)
def _sc_agg(y_hbm, ei_hbm, z0_hbm, out_hbm,
            src_v, dst_v, r0, r1, r2, z_sh,
            gs0, gs1, gs2, ss0, ss1, ss2):
    c = lax.axis_index("c")
    s = lax.axis_index("s")
    wid = c * NSUB + s
    row0 = s * SLAB
    pltpu.sync_copy(z0_hbm.at[pl.ds(row0, SLAB)], z_sh.at[pl.ds(row0, SLAB)])

    @pl.when(s == NSUB - 1)
    def _():
        pltpu.sync_copy(z0_hbm.at[pl.ds(TAIL0, TAILN)],
                        z_sh.at[pl.ds(TAIL0, TAILN)])

    plsc.subcore_barrier()

    rows = (r0, r1, r2)
    gsem = (gs0, gs1, gs2)
    ssem = (ss0, ss1, ss2)

    def fire_g(c, b):
        pltpu.async_copy(y_hbm.at[src_v.at[c]], rows[b], gsem[b])

    def wait_g(c, b):
        pltpu.make_async_copy(y_hbm.at[src_v.at[c]], rows[b], gsem[b]).wait()

    def fire_s(c, b):
        pltpu.async_copy(rows[b], z_sh.at[dst_v.at[c]], ssem[b], add=True)

    def wait_s(c, b):
        pltpu.make_async_copy(rows[b], z_sh.at[dst_v.at[c]], ssem[b]).wait()

    @pl.loop(0, NSB)
    def _(p):
        pltpu.sync_copy(ei_hbm.at[0, wid, p], src_v)
        pltpu.sync_copy(ei_hbm.at[1, wid, p], dst_v)
        fire_g(0, 0)
        fire_g(1, 1)
        wait_g(0, 0)
        fire_s(0, 0)
        fire_g(2, 2)

        @pl.loop(1, SB - 3, step=3)
        def _(t):
            for o, b in ((0, 1), (1, 2), (2, 0)):
                c = t + o
                wait_g(c, b)
                fire_s(c, b)
                wait_s(c - 1, (b - 1) % 3)
                fire_g(c + 2, (b + 2) % 3)

        wait_g(SB - 3, 1)
        fire_s(SB - 3, 1)
        wait_s(SB - 4, 0)
        fire_g(SB - 1, 0)
        wait_g(SB - 2, 2)
        fire_s(SB - 2, 2)
        wait_s(SB - 3, 1)
        wait_g(SB - 1, 0)
        fire_s(SB - 1, 0)
        wait_s(SB - 2, 2)
        wait_s(SB - 1, 0)

    plsc.subcore_barrier()
    pltpu.sync_copy(z_sh.at[pl.ds(row0, SLAB)],
                    out_hbm.at[c, pl.ds(row0, SLAB)])

    @pl.when(s == NSUB - 1)
    def _():
        pltpu.sync_copy(z_sh.at[pl.ds(TAIL0, TAILN)],
                        out_hbm.at[c, pl.ds(TAIL0, TAILN)])


@functools.partial(
    pl.kernel,
    out_type=jax.ShapeDtypeStruct((NW, NN), jnp.float32),
    mesh=_mesh,
    compiler_params=pltpu.CompilerParams(needs_layout_passes=False),
    scratch_types=[
        pltpu.VMEM((EW,), jnp.int32),
        pltpu.VMEM((NN,), jnp.float32),
    ],
)
def _sc_deg(ei_hbm, out_hbm, dst_v, cnt_v):
    c = lax.axis_index("c")
    s = lax.axis_index("s")
    wid = c * NSUB + s
    pltpu.sync_copy(ei_hbm.at[1, wid], dst_v)
    zero = jnp.zeros((16,), jnp.float32)

    @pl.loop(0, NN // 16)
    def _(i):
        cnt_v[pl.ds(i * 16, 16)] = zero

    ones = jnp.ones((16,), jnp.float32)

    @pl.loop(0, EW // 16)
    def _(i):
        idx = dst_v[pl.ds(i * 16, 16)]
        plsc.addupdate_scatter(cnt_v, [idx], ones)

    pltpu.sync_copy(cnt_v, out_hbm.at[wid])


def kernel(x, meta_x, edge_index, meta_edge_index, W_lin, b_lin, W_meta,
           b_meta, Wc0, bc0, Wc1, bc1, Wg, bg, Wcls, bcls):
    ei = edge_index.astype(jnp.int32).reshape(2, NW, NSB, SB, K)
    meta_dst = meta_edge_index[1].astype(jnp.int32).reshape(NN, 1)
    mx_pad = jnp.pad(meta_x, ((0, D - NMETA), (0, 0)))
    zeros_nd = jnp.zeros((NN, D), jnp.float32)

    deg_p = _sc_deg(edge_index.astype(jnp.int32).reshape(2, NW, EW))
    xw0, mh = _tc1(x, W_lin, b_lin.reshape(1, D), mx_pad, W_meta,
                   b_meta.reshape(1, D), Wc0)
    dinv, y0 = _tc2(deg_p, xw0)
    z0 = _sc_agg(y0, ei, zeros_nd)
    y1 = _tc3(z0, y0, dinv, bc0.reshape(1, D), Wc1)
    z1 = _sc_agg(y1, ei, zeros_nd)
    xgb = _tc4a(z1, y1, dinv, bc1.reshape(1, D), Wg)
    return _tc4b(xgb, mh, Wg, bg.reshape(1, D), meta_dst, Wcls,
                 bcls.reshape(1, NCLS))


# speedup vs baseline: 33.8227x; 1.0708x over previous
import functools

import jax
import jax.numpy as jnp
from jax import lax
from jax.experimental import pallas as pl
from jax.experimental.pallas import tpu as pltpu
from jax.experimental.pallas import tpu_sc as plsc

NN = 10000
NE = 320000
D = 128
NCLS = 16
NMETA = 100
NEG = 0.2

NCORE = 2
NSUB = 16
NW = NCORE * NSUB
EW = NE // NW
K = 80
NCHUNK = EW // K
SB = 25
NSB = NCHUNK // SB
SLAB = 624
TAIL0 = SLAB * NSUB
TAILN = NN - TAIL0

_P = lax.Precision.DEFAULT


def _lrelu(t):
    return jnp.where(t > 0, t, NEG * t)


def _tc1_body(x_ref, wl_ref, bl_ref, mx_ref, wm_ref, bm_ref, wc0_ref,
              xw0_ref, mh_ref):
    h = _lrelu(jnp.dot(x_ref[...], wl_ref[...], precision=_P) + bl_ref[...])
    xw0_ref[...] = jnp.dot(h, wc0_ref[...], precision=_P)
    mh_ref[...] = _lrelu(jnp.dot(mx_ref[...], wm_ref[...], precision=_P)
                         + bm_ref[...])


def _tc2_body(dp_ref, xw0_ref, dinv_ref, y0_ref):
    deg = lax.dot_general(dp_ref[...], jnp.ones((NW, 1), jnp.float32),
                          (((0,), (0,)), ((), ())), precision=_P) + 1.0
    dinv = lax.rsqrt(deg)
    dinv_ref[...] = dinv
    y0_ref[...] = dinv * xw0_ref[...]


def _tc3_body(z_ref, y0_ref, dinv_ref, bc0_ref, wc1_ref, y1_ref):
    dinv = dinv_ref[...]
    h1 = _lrelu(dinv * (z_ref[0] + z_ref[1] + y0_ref[...]) + bc0_ref[...])
    y1_ref[...] = dinv * jnp.dot(h1, wc1_ref[...], precision=_P)


def _tc4a_body(z_ref, y1_ref, dinv_ref, bc1_ref, wg_ref, xgb_ref):
    dinv = dinv_ref[...]
    h2 = _lrelu(dinv * (z_ref[0] + z_ref[1] + y1_ref[...]) + bc1_ref[...])
    xgb_ref[...] = jnp.dot(h2, wg_ref[...], precision=_P)


def _tc4b_body(xgb_ref, mh_ref, wg_ref, bg_ref, md_ref, wcls_ref, bcls_ref,
               out_ref):
    xgb = xgb_ref[...]
    xgm = jnp.dot(mh_ref[...], wg_ref[...], precision=_P)
    lanes = lax.broadcasted_iota(jnp.int32, (NN, D), 1) + NN
    p = jnp.where(md_ref[...] == lanes, 1.0, 0.0)
    s = lax.dot_general(p, xgb, (((0,), (0,)), ((), ())), precision=_P)
    cnt = lax.dot_general(p, jnp.ones((NN, 1), jnp.float32),
                          (((0,), (0,)), ((), ())), precision=_P)
    dinv_m = lax.rsqrt(cnt + 1.0)
    hm_b = _lrelu(xgb + bg_ref[...])
    hm_m = _lrelu(dinv_m * s + (dinv_m * dinv_m) * xgm + bg_ref[...])

    def logsoftmax(t):
        m = jnp.max(t, axis=1, keepdims=True)
        return t - m - jnp.log(jnp.sum(jnp.exp(t - m), axis=1, keepdims=True))

    out_ref[0:NN, :] = logsoftmax(jnp.dot(hm_b, wcls_ref[...], precision=_P)
                                  + bcls_ref[...])
    om = logsoftmax(jnp.dot(hm_m, wcls_ref[...], precision=_P)
                    + bcls_ref[...])
    out_ref[NN:NN + NMETA, :] = om[:NMETA]


def _f32(*shape):
    return jax.ShapeDtypeStruct(shape, jnp.float32)


_tc1 = pl.pallas_call(_tc1_body, out_shape=(_f32(NN, D), _f32(D, D)))
_tc2 = pl.pallas_call(_tc2_body, out_shape=(_f32(NN, 1), _f32(NN, D)))
_tc3 = pl.pallas_call(_tc3_body, out_shape=_f32(NN, D))
_tc4a = pl.pallas_call(_tc4a_body, out_shape=_f32(NN, D))
_tc4b = pl.pallas_call(_tc4b_body, out_shape=_f32(NN + NMETA, NCLS))


_mesh = plsc.VectorSubcoreMesh(core_axis_name="c", subcore_axis_name="s")


@functools.partial(
    pl.kernel,
    out_type=jax.ShapeDtypeStruct((NCORE, NN, D), jnp.float32),
    mesh=_mesh,
    scratch_types=[
        pltpu.VMEM((SB, K), jnp.int32),
        pltpu.VMEM((SB, K), jnp.int32),
        pltpu.VMEM((SB, K), jnp.int32),
        pltpu.VMEM((SB, K), jnp.int32),
        pltpu.VMEM((K, D), jnp.float32),
        pltpu.VMEM((K, D), jnp.float32),
        pltpu.VMEM((K, D), jnp.float32),
        pltpu.VMEM_SHARED((NN, D), jnp.float32),
        pltpu.SemaphoreType.DMA,
        pltpu.SemaphoreType.DMA,
        pltpu.SemaphoreType.DMA,
        pltpu.SemaphoreType.DMA,
        pltpu.SemaphoreType.DMA,
        pltpu.SemaphoreType.DMA,
        pltpu.SemaphoreType.DMA,
        pltpu.SemaphoreType.DMA,
        pltpu.SemaphoreType.DMA,
        pltpu.SemaphoreType.DMA,
    ],
)
def _sc_agg(y_hbm, ei_hbm, z0_hbm, out_hbm,
            src0_v, src1_v, dst0_v, dst1_v, r0, r1, r2, z_sh,
            gs0, gs1, gs2, ss0, ss1, ss2, x0s, x0d, x1s, x1d):
    c = lax.axis_index("c")
    s = lax.axis_index("s")
    wid = c * NSUB + s
    row0 = s * SLAB
    pltpu.sync_copy(z0_hbm.at[pl.ds(row0, SLAB)], z_sh.at[pl.ds(row0, SLAB)])

    @pl.when(s == NSUB - 1)
    def _():
        pltpu.sync_copy(z0_hbm.at[pl.ds(TAIL0, TAILN)],
                        z_sh.at[pl.ds(TAIL0, TAILN)])

    plsc.subcore_barrier()

    rows = (r0, r1, r2)
    gsem = (gs0, gs1, gs2)
    ssem = (ss0, ss1, ss2)
    srcb = (src0_v, src1_v)
    dstb = (dst0_v, dst1_v)
    xsem = ((x0s, x0d), (x1s, x1d))

    def stage(p):
        b = p % 2
        pltpu.async_copy(ei_hbm.at[0, wid, p], srcb[b], xsem[b][0])
        pltpu.async_copy(ei_hbm.at[1, wid, p], dstb[b], xsem[b][1])

    def stage_wait(p):
        b = p % 2
        pltpu.make_async_copy(ei_hbm.at[0, wid, p], srcb[b],
                              xsem[b][0]).wait()
        pltpu.make_async_copy(ei_hbm.at[1, wid, p], dstb[b],
                              xsem[b][1]).wait()

    def fire_g(b, o, rb):
        pltpu.async_copy(y_hbm.at[srcb[b].at[o]], rows[rb], gsem[rb])

    def wait_g(b, o, rb):
        pltpu.make_async_copy(y_hbm.at[srcb[b].at[o]], rows[rb],
                              gsem[rb]).wait()

    def fire_s(b, o, rb):
        pltpu.async_copy(rows[rb], z_sh.at[dstb[b].at[o]], ssem[rb],
                         add=True)

    def wait_s(b, o, rb):
        pltpu.make_async_copy(rows[rb], z_sh.at[dstb[b].at[o]],
                              ssem[rb]).wait()

    stage(0)
    stage_wait(0)
    fire_g(0, 0, 0)
    fire_g(0, 1, 1)

    for p in range(NSB):
        b = p % 2
        r = (p * SB) % 3
        wait_g(b, 0, r)
        fire_s(b, 0, r)
        if p > 0:
            wait_s(b, 0, (r - 1) % 3)
        fire_g(b, 2, (r + 2) % 3)
        if p + 1 < NSB:
            stage(p + 1)

        rr = ((r + 1) % 3, (r + 2) % 3, r)

        @pl.loop(1, SB - 3, step=3)
        def _(o):
            for t in range(3):
                rt = rr[t]
                wait_g(b, o + t, rt)
                fire_s(b, o + t, rt)
                wait_s(b, o + t, (rt - 1) % 3)
                fire_g(b, o + t + 2, (rt + 2) % 3)

        rb = (r + 22) % 3
        wait_g(b, SB - 3, rb)
        fire_s(b, SB - 3, rb)
        wait_s(b, SB - 3, (rb - 1) % 3)
        fire_g(b, SB - 1, (rb + 2) % 3)
        if p + 1 < NSB:
            stage_wait(p + 1)
        rb = (r + 23) % 3
        wait_g(b, SB - 2, rb)
        fire_s(b, SB - 2, rb)
        wait_s(b, SB - 2, (rb - 1) % 3)
        if p + 1 < NSB:
            fire_g(1 - b, 0, (rb + 2) % 3)
        rb = (r + 24) % 3
        wait_g(b, SB - 1, rb)
        fire_s(b, SB - 1, rb)
        wait_s(b, SB - 1, (rb - 1) % 3)
        if p + 1 < NSB:
            fire_g(1 - b, 1, (rb + 2) % 3)

    wait_s((NSB - 1) % 2, SB - 1, (NCHUNK - 1) % 3)

    plsc.subcore_barrier()
    pltpu.sync_copy(z_sh.at[pl.ds(row0, SLAB)],
                    out_hbm.at[c, pl.ds(row0, SLAB)])

    @pl.when(s == NSUB - 1)
    def _():
        pltpu.sync_copy(z_sh.at[pl.ds(TAIL0, TAILN)],
                        out_hbm.at[c, pl.ds(TAIL0, TAILN)])


@functools.partial(
    pl.kernel,
    out_type=jax.ShapeDtypeStruct((NW, NN), jnp.float32),
    mesh=_mesh,
    compiler_params=pltpu.CompilerParams(needs_layout_passes=False),
    scratch_types=[
        pltpu.VMEM((EW,), jnp.int32),
        pltpu.VMEM((NN,), jnp.float32),
    ],
)
def _sc_deg(ei_hbm, out_hbm, dst_v, cnt_v):
    c = lax.axis_index("c")
    s = lax.axis_index("s")
    wid = c * NSUB + s
    pltpu.sync_copy(ei_hbm.at[1, wid], dst_v)
    zero = jnp.zeros((16,), jnp.float32)

    @pl.loop(0, NN // 16)
    def _(i):
        cnt_v[pl.ds(i * 16, 16)] = zero

    ones = jnp.ones((16,), jnp.float32)

    @pl.loop(0, EW // 16)
    def _(i):
        idx = dst_v[pl.ds(i * 16, 16)]
        plsc.addupdate_scatter(cnt_v, [idx], ones)

    pltpu.sync_copy(cnt_v, out_hbm.at[wid])


def kernel(x, meta_x, edge_index, meta_edge_index, W_lin, b_lin, W_meta,
           b_meta, Wc0, bc0, Wc1, bc1, Wg, bg, Wcls, bcls):
    ei = edge_index.astype(jnp.int32).reshape(2, NW, NSB, SB, K)
    meta_dst = meta_edge_index[1].astype(jnp.int32).reshape(NN, 1)
    mx_pad = jnp.pad(meta_x, ((0, D - NMETA), (0, 0)))
    zeros_nd = jnp.zeros((NN, D), jnp.float32)

    deg_p = _sc_deg(edge_index.astype(jnp.int32).reshape(2, NW, EW))
    xw0, mh = _tc1(x, W_lin, b_lin.reshape(1, D), mx_pad, W_meta,
                   b_meta.reshape(1, D), Wc0)
    dinv, y0 = _tc2(deg_p, xw0)
    z0 = _sc_agg(y0, ei, zeros_nd)
    y1 = _tc3(z0, y0, dinv, bc0.reshape(1, D), Wc1)
    z1 = _sc_agg(y1, ei, zeros_nd)
    xgb = _tc4a(z1, y1, dinv, bc1.reshape(1, D), Wg)
    return _tc4b(xgb, mh, Wg, bg.reshape(1, D), meta_dst, Wcls,
                 bcls.reshape(1, NCLS))
